# baseline (speedup 1.0000x reference)
import functools
from typing import Sequence

import jax
import jax.numpy as jnp
from jax import lax
from jax.experimental import pallas as pl
from jax.experimental.pallas import tpu as pltpu
from jax.experimental.pallas import tpu_sc as plsc

N = 10000
E = 320000
G = 512
NP = 10112
SENT = N
NC, NS, L = 2, 16, 16
NW = NC * NS
K = 80
ROWS_PT = NP // NS
ZROWS = 79
F32 = jnp.float32
I32 = jnp.int32

_MESH = dict(core_axis_name="c", subcore_axis_name="s",
             num_cores=NC, num_subcores=NS)


def _lrelu(v, s):
    return jnp.maximum(v, s * v)


def _lane_total(tv, red):
    for sh in (8, 4, 2, 1):
        red[pl.ds(0, 16)] = tv
        tv = tv + red[pl.ds(sh, 16)]
    return tv[0]


def _sc_edge_e(src, dst, xl_cs, xr_cs, we_cs, att_pad, c_tab):
    C = len(xl_cs)
    Fcs = tuple(int(t.shape[1]) for t in xl_cs)
    ipw = E // NW
    nchunks = ipw // K

    scratch = [
        pltpu.VMEM((K,), I32),
        pltpu.VMEM((K,), I32),
        pltpu.VMEM((K,), F32),
        pltpu.VMEM((K,), F32),
        pltpu.VMEM((C * 128,), F32),
        pltpu.VMEM((NP,), F32),
        pltpu.VMEM((32,), F32),
    ]
    fc_kinds = sorted(set(Fcs))
    buf_of = {fc: i for i, fc in enumerate(fc_kinds)}
    for fc in fc_kinds:
        scratch.append(pltpu.VMEM((K, fc), F32))
        scratch.append(pltpu.VMEM((K, fc), F32))
        scratch.append(pltpu.VMEM((K, fc), F32))

    def body(src_r, dst_r, *rest):
        xl_rs = rest[0:C]
        xr_rs = rest[C:2 * C]
        we_rs = rest[2 * C:3 * C]
        att_r = rest[3 * C]
        c_r = rest[3 * C + 1]
        e_out = rest[3 * C + 2]
        (idx_s, idx_d, lg_v, e_v, att_v, c_v, red) = rest[3 * C + 3:3 * C + 10]
        rowbufs = rest[3 * C + 10:]

        cid = lax.axis_index("c")
        sid = lax.axis_index("s")
        wid = sid * NC + cid
        base0 = wid * ipw
        pltpu.sync_copy(att_r, att_v)
        pltpu.sync_copy(c_r, c_v)
        red[pl.ds(16, 16)] = jnp.zeros((16,), F32)

        def chunk(j, carry):
            eb = base0 + j * K
            pltpu.sync_copy(src_r.at[pl.ds(eb, K)], idx_s)
            pltpu.sync_copy(dst_r.at[pl.ds(eb, K)], idx_d)
            for t in range(K // 16):
                lg_v[pl.ds(t * 16, 16)] = jnp.zeros((16,), F32)
            iot = lax.iota(I32, 16)
            for ci in range(C):
                fc = Fcs[ci]
                b = buf_of[fc]
                xlb = rowbufs[3 * b]
                xrb = rowbufs[3 * b + 1]
                web = rowbufs[3 * b + 2]
                pltpu.sync_copy(xl_rs[ci].at[idx_s], xlb)
                pltpu.sync_copy(xr_rs[ci].at[idx_d], xrb)
                pltpu.sync_copy(we_rs[ci].at[pl.ds(eb, K)], web)

                def tgroup(t, _):
                    r0 = t * 16
                    acc = lg_v[pl.ds(r0, 16)]
                    for kk in range(16):
                        tv = jnp.zeros((16,), F32)
                        for f in range(fc // 16):
                            v = (xlb[r0 + kk, pl.ds(f * 16, 16)]
                                 + xrb[r0 + kk, pl.ds(f * 16, 16)]
                                 + web[r0 + kk, pl.ds(f * 16, 16)])
                            v = jnp.maximum(v, 0.2 * v)
                            tv = tv + v * att_v[pl.ds(ci * 128 + f * 16, 16)]
                        acc = acc + jnp.where(iot == kk,
                                              _lane_total(tv, red), 0.0)
                    lg_v[pl.ds(r0, 16)] = acc
                    return 0

                lax.fori_loop(0, K // 16, tgroup, 0)
            for t in range(K // 16):
                dstv = idx_d[pl.ds(t * 16, 16)]
                cg = plsc.load_gather(c_v, [dstv])
                e_v[pl.ds(t * 16, 16)] = jnp.exp(lg_v[pl.ds(t * 16, 16)] - cg)
            pltpu.sync_copy(e_v, e_out.at[pl.ds(eb, K)])
            return carry

        lax.fori_loop(0, nchunks, chunk, 0)

    kern = pl.kernel(
        body,
        out_type=jax.ShapeDtypeStruct((E,), F32),
        mesh=plsc.VectorSubcoreMesh(**_MESH),
        scratch_types=scratch,
        compiler_params=pltpu.CompilerParams(needs_layout_passes=False, use_tc_tiling_on_sc=False),
        name=f"sc_edge_e_C{C}_F{Fcs[0]}",
    )
    return kern(src, dst, *xl_cs, *xr_cs, *we_cs, att_pad, c_tab)


def _sc_agg(src, dst, table, scale=None, with_den=False, n_items=E):
    Fc = int(table.shape[1])
    ipw = n_items // NW
    nchunks = ipw // K
    assert ipw % K == 0 and n_items % NW == 0

    scratch = [
        pltpu.VMEM((K,), I32),
        pltpu.VMEM((K,), I32),
        pltpu.VMEM((K,), F32),
        pltpu.VMEM((K, Fc), F32),
        pltpu.VMEM((K, 16), F32),
        pltpu.VMEM((ZROWS, Fc), F32),
        pltpu.VMEM((ZROWS, 16), F32),
        pltpu.VMEM_SHARED((NP, Fc), F32),
        pltpu.VMEM_SHARED((NP, 16), F32),
    ]
    out_type = [jax.ShapeDtypeStruct((2, NP, Fc), F32),
                jax.ShapeDtypeStruct((2, NP, 16), F32)]

    def body(*args):
        if scale is not None:
            (src_r, dst_r, tab_r, sc_r, out_r, den_r,
             idx_s, idx_d, sc_v, rows, den_b, zb, zb16, acc_sp, den_sp) = args
        else:
            (src_r, dst_r, tab_r, out_r, den_r,
             idx_s, idx_d, sc_v, rows, den_b, zb, zb16, acc_sp, den_sp) = args
            sc_r = None
        cid = lax.axis_index("c")
        sid = lax.axis_index("s")
        wid = sid * NC + cid
        base0 = wid * ipw

        def zrow(r, _):
            for f in range(Fc // 16):
                zb[r, pl.ds(f * 16, 16)] = jnp.zeros((16,), F32)
            zb16[r, pl.ds(0, 16)] = jnp.zeros((16,), F32)
            return 0
        lax.fori_loop(0, ZROWS, zrow, 0)
        for t in range(ROWS_PT // ZROWS):
            off = sid * ROWS_PT + t * ZROWS
            pltpu.sync_copy(zb, acc_sp.at[pl.ds(off, ZROWS)])
            if with_den:
                pltpu.sync_copy(zb16, den_sp.at[pl.ds(off, ZROWS)])
        if scale is None and with_den:
            lane0 = (lax.iota(I32, 16) == 0).astype(F32)
            def drow(r, _):
                den_b[r, pl.ds(0, 16)] = lane0
                return 0
            lax.fori_loop(0, K, drow, 0)
        plsc.subcore_barrier()

        def chunk(j, carry):
            eb = base0 + j * K
            pltpu.sync_copy(src_r.at[pl.ds(eb, K)], idx_s)
            pltpu.sync_copy(dst_r.at[pl.ds(eb, K)], idx_d)
            pltpu.sync_copy(tab_r.at[idx_s], rows)
            if sc_r is not None:
                pltpu.sync_copy(sc_r.at[pl.ds(eb, K)], sc_v)
                lane0 = (lax.iota(I32, 16) == 0).astype(F32)

                def tgroup(t, _):
                    r0 = t * 16
                    sv = sc_v[pl.ds(r0, 16)]
                    for kk in range(16):
                        s = sv[kk]
                        for f in range(Fc // 16):
                            rows[r0 + kk, pl.ds(f * 16, 16)] = (
                                rows[r0 + kk, pl.ds(f * 16, 16)] * s)
                        if with_den:
                            den_b[r0 + kk, pl.ds(0, 16)] = lane0 * s
                    return 0

                lax.fori_loop(0, K // 16, tgroup, 0)
            pltpu.sync_copy(rows, acc_sp.at[idx_d], add=True)
            if with_den:
                pltpu.sync_copy(den_b, den_sp.at[idx_d], add=True)
            return carry

        lax.fori_loop(0, nchunks, chunk, 0)
        plsc.subcore_barrier()
        for t in range(ROWS_PT // ZROWS):
            off = sid * ROWS_PT + t * ZROWS
            pltpu.sync_copy(acc_sp.at[pl.ds(off, ZROWS)],
                            out_r.at[cid, pl.ds(off, ZROWS)])
            if with_den:
                pltpu.sync_copy(den_sp.at[pl.ds(off, ZROWS)],
                                den_r.at[cid, pl.ds(off, ZROWS)])

    kern = pl.kernel(
        body,
        out_type=out_type,
        mesh=plsc.VectorSubcoreMesh(**_MESH),
        scratch_types=scratch,
        compiler_params=pltpu.CompilerParams(needs_layout_passes=False, use_tc_tiling_on_sc=False),
        name=f"sc_agg_F{Fc}{'_den' if with_den else ''}",
    )
    args = (src, dst, table) + ((scale,) if scale is not None else ())
    out, den = kern(*args)
    return out, den


def _sc_pools(a, starts):
    F = int(a.shape[1])
    GPT = G // NW

    scratch = [
        pltpu.VMEM((528,), I32),
        pltpu.VMEM((16,), I32),
        pltpu.VMEM((16, F), F32),
        pltpu.VMEM((F,), F32),
        pltpu.VMEM((F,), F32),
    ]

    def body(a_r, st_r, out_r, st_v, idx_b, rows, sm_v, mx_v):
        cid = lax.axis_index("c")
        sid = lax.axis_index("s")
        wid = sid * NC + cid
        pltpu.sync_copy(st_r, st_v)
        iot = lax.iota(I32, 16)

        def graph(gi, _):
            g = wid * GPT + gi
            sv = st_v[pl.ds(g, 16)]
            s = sv[0]
            e = sv[1]
            for f in range(F // 16):
                sm_v[pl.ds(f * 16, 16)] = jnp.zeros((16,), F32)
                mx_v[pl.ds(f * 16, 16)] = jnp.full((16,), -3e38, F32)
            nch = (e - s + 15) // 16

            def jb(j, _):
                base = s + j * 16
                idx_b[pl.ds(0, 16)] = jnp.minimum(base + iot, e - 1)
                pltpu.sync_copy(a_r.at[idx_b], rows)

                def rb(r, _):
                    valid = (base + r) < e
                    vf = jnp.where(valid, 1.0, 0.0).astype(F32)
                    big = jnp.where(valid, 0.0, -3e38).astype(F32)
                    for f in range(F // 16):
                        row = rows[r, pl.ds(f * 16, 16)]
                        sm_v[pl.ds(f * 16, 16)] = (
                            sm_v[pl.ds(f * 16, 16)] + row * vf)
                        mx_v[pl.ds(f * 16, 16)] = jnp.maximum(
                            mx_v[pl.ds(f * 16, 16)], row + big)
                    return 0

                lax.fori_loop(0, 16, rb, 0)
                return 0

            lax.fori_loop(0, nch, jb, 0)
            pltpu.sync_copy(sm_v, out_r.at[g, pl.ds(0, F)])
            pltpu.sync_copy(mx_v, out_r.at[g, pl.ds(F, F)])
            return 0

        lax.fori_loop(0, GPT, graph, 0)

    kern = pl.kernel(
        body,
        out_type=jax.ShapeDtypeStruct((G, 2 * F), F32),
        mesh=plsc.VectorSubcoreMesh(**_MESH),
        scratch_types=scratch,
        compiler_params=pltpu.CompilerParams(needs_layout_passes=False, use_tc_tiling_on_sc=False),
        name=f"sc_pools_F{F}",
    )
    return kern(a, starts)


def _tc_matmul(x, w, b=None, act=None, block_m=128, out_chunks=None):
    m, kd = x.shape
    o = w.shape[1]
    assert m % block_m == 0
    grid = (m // block_m,)
    if out_chunks:
        widths = out_chunks
        offs = [sum(widths[:i]) for i in range(len(widths))]
        out_shape = [jax.ShapeDtypeStruct((m, f), F32) for f in widths]
        out_specs = [pl.BlockSpec((block_m, f), lambda i: (i, 0))
                     for f in widths]
    else:
        out_shape = jax.ShapeDtypeStruct((m, o), F32)
        out_specs = pl.BlockSpec((block_m, o), lambda i: (i, 0))

    has_b = b is not None

    def body(*refs):
        x_r = refs[0]
        w_r = refs[1]
        i = 2
        b_r = None
        if has_b:
            b_r = refs[2]
            i = 3
        outs = refs[i:]
        acc = jnp.dot(x_r[...], w_r[...], preferred_element_type=F32)
        if has_b:
            acc = acc + b_r[...]
        if act == "lrelu":
            acc = _lrelu(acc, 0.01)
        if out_chunks:
            for oi, (f, of) in enumerate(zip(widths, offs)):
                outs[oi][...] = acc[:, of:of + f]
        else:
            outs[0][...] = acc

    in_specs = [pl.BlockSpec((block_m, kd), lambda i: (i, 0)),
                pl.BlockSpec((kd, o), lambda i: (0, 0))]
    args = [x, w]
    if has_b:
        in_specs.append(pl.BlockSpec((1, o), lambda i: (0, 0)))
        args.append(b.reshape(1, o))
    return pl.pallas_call(
        body, grid=grid, in_specs=in_specs, out_specs=out_specs,
        out_shape=out_shape)(*args)


def _u_spec(fc):
    return pl.BlockSpec((1, 128, fc), lambda i: (0, i, 0))


def _u_spec1(fc):
    return pl.BlockSpec((1, 128, fc), lambda i: (1, i, 0))


def _tc_c_tab(xl_cs, xr_cs, att, la=None, we=None):
    o = sum(t.shape[1] for t in xl_cs)
    C = len(xl_cs)
    widths = [t.shape[1] for t in xl_cs]
    offs = [sum(widths[:i]) for i in range(C)]
    has_la = la is not None

    def body(*refs):
        i = 0
        xls = refs[:C]; i = C
        xrs = refs[i:i + C]; i += C
        att_r = refs[i]; i += 1
        if has_la:
            la0 = refs[i]; la1 = refs[i + 1]; we_r = refs[i + 2]; i += 3
        out = refs[i]
        if has_la:
            las = la0[0] + la1[0]
            cnt = las[:, 11]
            lat = las[:, :11] / jnp.clip(cnt, 1.0)[:, None]
            lw = jnp.dot(lat, we_r[...], preferred_element_type=F32)
        acc = jnp.zeros((xls[0].shape[0],), F32)
        for ci in range(C):
            v = xls[ci][...] + xrs[ci][...]
            if has_la:
                v = v + lw[:, offs[ci]:offs[ci] + widths[ci]]
            v = _lrelu(v, 0.2)
            acc = acc + jnp.sum(v * att_r[0, offs[ci]:offs[ci] + widths[ci]][None, :],
                                axis=1)
        out[0, :] = acc

    in_specs = ([pl.BlockSpec((128, f), lambda i: (i, 0)) for f in widths]
                + [pl.BlockSpec((128, f), lambda i: (i, 0)) for f in widths]
                + [pl.BlockSpec((1, o), lambda i: (0, 0))])
    args = list(xl_cs) + list(xr_cs) + [att.reshape(1, o)]
    if has_la:
        in_specs += [_u_spec(16), _u_spec1(16),
                     pl.BlockSpec((11, o), lambda i: (0, 0))]
        args += [la, la, we]
    return pl.pallas_call(
        body, grid=(NP // 128,), in_specs=in_specs,
        out_specs=pl.BlockSpec((1, 128), lambda i: (0, i)),
        out_shape=jax.ShapeDtypeStruct((1, NP), F32))(*args)


def _tc_gat_fin(u, den, xl_cs, bias, self_loops, prev=None, post_lrelu=False):
    C = len(xl_cs)
    widths = [t.shape[1] for t in xl_cs]
    offs = [sum(widths[:i]) for i in range(C)]
    o = sum(widths)
    has_prev = prev is not None

    def body(*refs):
        i = 0
        us = refs[:C]; i = C
        d0 = refs[i]; d1 = refs[i + 1]; i += 2
        xls = refs[i:i + C]; i += C
        b_r = refs[i]; i += 1
        p_r = None
        if has_prev:
            p_r = refs[i]; i += 1
        out = refs[i]
        den_v = d0[0][:, 0] + d1[0][:, 0]
        if self_loops:
            den_v = den_v + 1.0
        den_v = jnp.clip(den_v, 1e-16)
        for ci in range(C):
            ut = us[ci][0] + us[ci][1]
            if self_loops:
                ut = ut + xls[ci][...]
            y = ut / den_v[:, None] + b_r[0, offs[ci]:offs[ci] + widths[ci]][None, :]
            if has_prev:
                y = y + p_r[:, offs[ci]:offs[ci] + widths[ci]]
            if post_lrelu:
                y = _lrelu(y, 0.01)
            out[:, offs[ci]:offs[ci] + widths[ci]] = y

    in_specs = ([pl.BlockSpec((2, 128, f), lambda i: (0, i, 0)) for f in widths]
                + [_u_spec(16), _u_spec1(16)]
                + [pl.BlockSpec((128, f), lambda i: (i, 0)) for f in widths]
                + [pl.BlockSpec((1, o), lambda i: (0, 0))])
    args = list(u) + [den, den] + list(xl_cs) + [bias.reshape(1, o)]
    if has_prev:
        in_specs.append(pl.BlockSpec((128, o), lambda i: (i, 0)))
        args.append(prev)
    return pl.pallas_call(
        body, grid=(NP // 128,), in_specs=in_specs,
        out_specs=pl.BlockSpec((128, o), lambda i: (i, 0)),
        out_shape=jax.ShapeDtypeStruct((NP, o), F32))(*args)


def _tc_score_scale(h, w):
    def body(h_r, w_r, sc_out, hs_out):
        wv = w_r[0]
        nrm = jnp.sqrt(jnp.sum(wv * wv))
        s = jnp.tanh(jnp.dot(h_r[...], (wv / nrm)[:, None],
                             preferred_element_type=F32))[:, 0]
        sc_out[0, :] = s
        hs_out[...] = h_r[...] * s[:, None]

    o = h.shape[1]
    return pl.pallas_call(
        body, grid=(NP // 128,),
        in_specs=[pl.BlockSpec((128, o), lambda i: (i, 0)),
                  pl.BlockSpec((1, o), lambda i: (0, 0))],
        out_specs=[pl.BlockSpec((1, 128), lambda i: (0, i)),
                   pl.BlockSpec((128, o), lambda i: (i, 0))],
        out_shape=[jax.ShapeDtypeStruct((1, NP), F32),
                   jax.ShapeDtypeStruct((NP, o), F32)])(h, w.reshape(1, o))


def _tc_combine(u):
    F = u.shape[2]

    def body(u_r, out):
        out[...] = u_r[0] + u_r[1]

    return pl.pallas_call(
        body, grid=(NP // 128,),
        in_specs=[pl.BlockSpec((2, 128, F), lambda i: (0, i, 0))],
        out_specs=pl.BlockSpec((128, F), lambda i: (i, 0)),
        out_shape=jax.ShapeDtypeStruct((NP, F), F32))(u)


def _tc_sage_fin(u, cnt_den, hp, wl, bl, wr):
    o = hp.shape[1]

    def body(u_r, d0, d1, hp_r, wl_r, wr_r, b_r, out):
        cnt = jnp.clip(d0[0][:, 0] + d1[0][:, 0], 1.0)
        agg = (u_r[0] + u_r[1]) / cnt[:, None]
        acc = (jnp.dot(agg, wl_r[...], preferred_element_type=F32)
               + jnp.dot(hp_r[...], wr_r[...], preferred_element_type=F32)
               + b_r[...])
        out[...] = _lrelu(hp_r[...] + acc, 0.01)

    return pl.pallas_call(
        body, grid=(NP // 128,),
        in_specs=[pl.BlockSpec((2, 128, o), lambda i: (0, i, 0)),
                  _u_spec(16), _u_spec1(16),
                  pl.BlockSpec((128, o), lambda i: (i, 0)),
                  pl.BlockSpec((o, o), lambda i: (0, 0)),
                  pl.BlockSpec((o, o), lambda i: (0, 0)),
                  pl.BlockSpec((1, o), lambda i: (0, 0))],
        out_specs=pl.BlockSpec((128, o), lambda i: (i, 0)),
        out_shape=jax.ShapeDtypeStruct((NP, o), F32))(
            u, cnt_den, cnt_den, hp, wl, wr, bl.reshape(1, o))


def _tc_branch_fin(p1, p2, counts):
    F2 = p1.shape[1]
    F = F2 // 2

    def body(p1_r, p2_r, c_r, out):
        cnt = c_r[...]
        cc = jnp.clip(cnt, 1.0)
        pos = cnt > 0
        acc = jnp.zeros_like(p1_r[...])
        for p_r in (p1_r, p2_r):
            mean = p_r[:, :F] / cc
            mx = jnp.where(pos, p_r[:, F:], 0.0)
            acc = acc + jnp.concatenate([mean, mx], axis=1)
        out[...] = acc

    return pl.pallas_call(
        body, grid=(G // 128,),
        in_specs=[pl.BlockSpec((128, F2), lambda i: (i, 0)),
                  pl.BlockSpec((128, F2), lambda i: (i, 0)),
                  pl.BlockSpec((128, 1), lambda i: (i, 0))],
        out_specs=pl.BlockSpec((128, F2), lambda i: (i, 0)),
        out_shape=jax.ShapeDtypeStruct((G, F2), F32))(p1, p2, counts)


def _tc_gcn(v, adj, w, b):
    nf = adj.shape[0]

    def body(v_r, a_r, w_r, b_r, out):
        A = a_r[...] + jnp.eye(nf, dtype=F32)
        d = jax.lax.rsqrt(jnp.sum(A, axis=1))
        An = d[:, None] * A * d[None, :]
        h = jax.lax.dot_general(An, v_r[...], (((1,), (1,)), ((), ())),
                                preferred_element_type=F32)
        out[...] = jax.lax.dot_general(h, w_r[...], (((0,), (0,)), ((), ())),
                                       preferred_element_type=F32) + b_r[...]

    o = w.shape[1]
    return pl.pallas_call(
        body,
        out_shape=jax.ShapeDtypeStruct((G, o), F32))(
            v, adj, w, b.reshape(1, o))


def _chunks_of(o):
    ws = []
    r = o
    while r > 0:
        ws.append(min(128, r))
        r -= 128
    return ws


def _pad_att(att, widths):
    segs = []
    off = 0
    for wd in widths:
        segs.append(jnp.pad(att[off:off + wd], (0, 128 - wd)))
        off += wd
    return jnp.concatenate(segs)


def _gat_layer(x, src, dst, we_full, la, p, self_loops, prev=None,
               post_lrelu=False, e_scale_tabs=None):
    o = p['Wl'].shape[1]
    widths = _chunks_of(o)
    wlr = jnp.concatenate([p['Wl'], p['Wr']], axis=1)
    blr = jnp.concatenate([p['bl'], p['br']])
    outs = _tc_matmul(x, wlr, blr, out_chunks=widths + widths)
    xl_cs = tuple(outs[:len(widths)])
    xr_cs = tuple(outs[len(widths):])
    we_cs = tuple(_tc_matmul(we_full, p['We'], None, block_m=512,
                             out_chunks=widths))
    c_tab = _tc_c_tab(xl_cs, xr_cs, p['att'],
                      la=la if self_loops else None,
                      we=p['We'] if self_loops else None)
    c_tab = c_tab.reshape(NP)
    att_pad = _pad_att(p['att'], widths)
    e = _sc_edge_e(src, dst, xl_cs, xr_cs, we_cs, att_pad, c_tab)
    us = []
    den = None
    for ci, xlc in enumerate(xl_cs):
        u, d = _sc_agg(src, dst, xlc, scale=e, with_den=(ci == 0))
        us.append(u)
        if ci == 0:
            den = d
    return _tc_gat_fin(us, den, xl_cs, p['bias'], self_loops,
                       prev=prev, post_lrelu=post_lrelu)


def _branch(hs, sp, dp, we_full_cache, la_p, bp_starts, counts_p, c1, c2,
            residual_first):
    a = _gat_layer(hs, sp, dp, we_full_cache, la_p, c1, True,
                   prev=hs if residual_first else None, post_lrelu=True)
    p1 = _sc_pools(a, bp_starts)
    b = _gat_layer(a, sp, dp, we_full_cache, la_p, c2, True,
                   prev=a, post_lrelu=True)
    p2 = _sc_pools(b, bp_starts)
    return _tc_branch_fin(p1, p2, counts_p)


def kernel(x, edge_attr, params, edge_index, batch):
    src = edge_index[0].astype(I32)
    dst = edge_index[1].astype(I32)
    x_pad = jnp.concatenate([x, jnp.zeros((NP - N, x.shape[1]), F32)], 0)
    ea16 = jnp.pad(edge_attr, ((0, 0), (0, 5)))
    ea16 = ea16.at[:, 11].set(1.0)
    iota_e = jnp.arange(E, dtype=I32)

    la_sh, _ = _sc_agg(iota_e, dst, ea16, scale=None, with_den=False)

    h = _gat_layer(x_pad, src, dst, edge_attr, None, params['sh1'], False,
                   post_lrelu=True)
    h = _gat_layer(h, src, dst, edge_attr, la_sh, params['sh2'], True,
                   post_lrelu=True)

    score2d, hs_full = _tc_score_scale(h, params['pool_w'])
    score = score2d.reshape(NP)[:N]

    n = N
    g = G
    batch = batch.astype(I32)
    order = jnp.lexsort((-score, batch))
    counts = jax.ops.segment_sum(jnp.ones(n, I32), batch, num_segments=g)
    kk = (4 * counts + 4) // 5
    starts = jnp.cumsum(counts) - counts
    bs = batch[order]
    rank = jnp.arange(n) - starts[bs]
    keep = rank < kk[bs]
    new_pos = jnp.cumsum(keep.astype(I32)) - 1
    new_idx = jnp.full(n, -1, I32).at[order].set(jnp.where(keep, new_pos, -1))
    sidx = jnp.argsort(jnp.logical_not(keep))
    perm_full = order[sidx]
    n_keep = jnp.sum(keep.astype(I32))
    em = (new_idx[src] >= 0) & (new_idx[dst] >= 0)
    sp = jnp.where(em, new_idx[src], SENT).astype(I32)
    dp = jnp.where(em, new_idx[dst], SENT).astype(I32)
    bp = jnp.where(jnp.arange(n) < n_keep, batch[perm_full], g).astype(I32)
    bp_starts = jnp.searchsorted(bp, jnp.arange(513, dtype=I32),
                                 side='left').astype(I32)
    bp_starts = jnp.pad(bp_starts, (0, 15))
    counts_p = (bp_starts[1:513] - bp_starts[:512]).astype(F32).reshape(G, 1)

    perm_pad = jnp.concatenate([perm_full.astype(I32),
                                jnp.full((10240 - n,), SENT, I32)])
    dst_iota = jnp.concatenate([jnp.arange(n, dtype=I32),
                                jnp.full((10240 - n,), SENT, I32)])
    hp_parts = []
    for ci in range(2):
        u, _ = _sc_agg(perm_pad, dst_iota, hs_full[:, ci * 128:(ci + 1) * 128],
                       scale=None, with_den=False, n_items=10240)
        hp_parts.append(_tc_combine(u))
    hp = jnp.concatenate(hp_parts, axis=1)

    la_p, _ = _sc_agg(iota_e, dp, ea16, scale=None, with_den=False)

    su0, scnt = _sc_agg(sp, dp, hp[:, :128], scale=None, with_den=True)
    su1, _ = _sc_agg(sp, dp, hp[:, 128:], scale=None, with_den=False)
    su = jnp.concatenate([su0, su1], axis=2)
    hs = _tc_sage_fin(su, scnt, hp, params['sage']['Wl'],
                      params['sage']['bl'], params['sage']['Wr'])

    gc = _branch(hs, sp, dp, edge_attr, la_p, bp_starts, counts_p,
                 params['bc1'], params['bc2'], True)
    gc = _tc_matmul(gc, params['bc_l1']['W'], params['bc_l1']['b'],
                    act="lrelu")
    gc = _tc_matmul(gc, params['bc_l2']['W'], params['bc_l2']['b'],
                    act="lrelu")
    out_coarse = _tc_matmul(gc, params['bc_l3']['W'], params['bc_l3']['b'])

    gf = _branch(hs, sp, dp, edge_attr, la_p, bp_starts, counts_p,
                 params['bf1'], params['bf2'], True)
    gf = _tc_matmul(gf, params['bf_l1']['W'], params['bf_l1']['b'],
                    act="lrelu")
    out_fine = _tc_matmul(gf, params['bf_l2']['W'], params['bf_l2']['b'])
    out_fine = jnp.concatenate([out_fine, out_coarse], axis=-1)
    out_fine = _tc_gcn(out_fine, params['adj_c'], params['gcn1']['W'],
                       params['gcn1']['b'])

    g2 = _branch(hs, sp, dp, edge_attr, la_p, bp_starts, counts_p,
                 params['bg1'], params['bg2'], False)
    g2 = _tc_matmul(g2, params['bg_l1']['W'], params['bg_l1']['b'],
                    act="lrelu")
    out_fine2 = _tc_matmul(g2, params['bg_l2']['W'], params['bg_l2']['b'])
    out_fine2 = jnp.concatenate([out_fine2, out_fine], axis=-1)
    out_fine2 = _tc_gcn(out_fine2, params['adj_f'], params['gcn2']['W'],
                        params['gcn2']['b'])
    return (out_coarse, out_fine, out_fine2)
